# baseline (speedup 1.0000x reference)
import functools

import jax
import jax.numpy as jnp
from jax import lax
from jax.experimental import pallas as pl
from jax.experimental.pallas import tpu as pltpu
from jax.experimental.pallas import tpu_sc as plsc

_N = 10000
_E = 320000
_H = 128
_NSEG = 64
_CHUNK = 96
_EPW = _E // 16
_NCHUNK = -(-_EPW // _CHUNK)
_EPAD = _NCHUNK * _CHUNK

_STRIPE = 632
_LAST_OFF = 15 * _STRIPE
_LAST_STRIPE = _N - _LAST_OFF

_BN = 1000
_NB = (2 * _N) // _BN


@functools.partial(
    pl.kernel,
    mesh=plsc.VectorSubcoreMesh(core_axis_name="c", subcore_axis_name="s"),
    out_type=jax.ShapeDtypeStruct((2 * _N, _H), jnp.float32),
    scratch_types=[
        pltpu.VMEM((2, _CHUNK), jnp.int32),
        pltpu.VMEM((2, _CHUNK), jnp.int32),
        pltpu.VMEM((2, _CHUNK), jnp.int32),
        pltpu.VMEM((2, _CHUNK), jnp.int32),
        pltpu.VMEM((_CHUNK, _H), jnp.float32),
        pltpu.VMEM((_CHUNK, _H), jnp.float32),
        pltpu.VMEM_SHARED((_N + 8, _H), jnp.float32),
        pltpu.SemaphoreType.DMA,
        pltpu.SemaphoreType.DMA,
        pltpu.SemaphoreType.DMA,
        pltpu.SemaphoreType.DMA,
        pltpu.SemaphoreType.DMA,
        pltpu.SemaphoreType.DMA,
        pltpu.SemaphoreType.DMA,
        pltpu.SemaphoreType.DMA,
    ],
)
def _sc_segsum(y_hbm, ei_hbm, zeros_hbm, out_hbm,
               ix0, ix1, ix2, ix3, rows0, rows1, acc,
               si0, si1, si2, si3, sg0, sg1, ss0, ss1):
    cid = lax.axis_index("c")
    sid = lax.axis_index("s")
    wid = cid * 16 + sid
    base = wid * _NCHUNK
    ix = [ix0, ix1, ix2, ix3]
    si = [si0, si1, si2, si3]
    rb = [rows0, rows1]
    sg = [sg0, sg1]
    ss = [ss0, ss1]

    @pl.when(sid < 15)
    def _():
        pltpu.sync_copy(zeros_hbm.at[pl.ds(sid * _STRIPE, _STRIPE)],
                        acc.at[pl.ds(sid * _STRIPE, _STRIPE)])

    @pl.when(sid == 15)
    def _():
        pltpu.sync_copy(zeros_hbm.at[pl.ds(_LAST_OFF, _LAST_STRIPE)],
                        acc.at[pl.ds(_LAST_OFF, _LAST_STRIPE)])

    plsc.subcore_barrier()

    def idx_start(c, k):
        pltpu.make_async_copy(ei_hbm.at[base + c], ix[k], si[k]).start()

    def idx_wait(c, k):
        pltpu.make_async_copy(ei_hbm.at[base + c], ix[k], si[k]).wait()

    def gather_start(k, r):
        pltpu.make_async_copy(y_hbm.at[ix[k].at[0]], rb[r], sg[r]).start()

    def gather_wait(k, r):
        pltpu.make_async_copy(y_hbm.at[ix[k].at[0]], rb[r], sg[r]).wait()

    def scat_start(k, r):
        pltpu.make_async_copy(rb[r], acc.at[ix[k].at[1]],
                              ss[r]).start(add=True)

    def scat_wait(k, r):
        pltpu.make_async_copy(rb[r], acc.at[ix[k].at[1]], ss[r]).wait()

    idx_start(0, 0)
    idx_start(1, 1)
    idx_wait(0, 0); idx_start(2, 2); gather_start(0, 0)
    idx_wait(1, 1); idx_start(3, 3); gather_start(1, 1)
    gather_wait(0, 0); scat_start(0, 0)
    idx_wait(2, 2); scat_wait(0, 0); idx_start(4, 0); gather_start(2, 0)
    gather_wait(1, 1); scat_start(1, 1)
    idx_wait(3, 3); scat_wait(1, 1); idx_start(5, 1); gather_start(3, 1)
    gather_wait(2, 0); scat_start(2, 0)

    def body(m, carry):
        c = 4 * m
        for ki in range(4):
            kr = ki & 1
            idx_wait(c + ki, ki)
            scat_wait(ki - 2 & 3, kr)
            idx_start(c + ki + 2, ki + 2 & 3)
            gather_start(ki, kr)
            gather_wait(ki - 1 & 3, kr ^ 1)
            scat_start(ki - 1 & 3, kr ^ 1)
        return carry

    lax.fori_loop(1, (_NCHUNK - 9) // 4 + 1, body, 0)
    e = _NCHUNK - 5
    idx_wait(e, 0); scat_wait(2, 0); idx_start(e + 2, 2); gather_start(0, 0)
    gather_wait(3, 1); scat_start(3, 1)
    idx_wait(e + 1, 1); scat_wait(3, 1); idx_start(e + 3, 3); gather_start(1, 1)
    gather_wait(0, 0); scat_start(0, 0)
    idx_wait(e + 2, 2); scat_wait(0, 0); idx_start(e + 4, 0); gather_start(2, 0)
    gather_wait(1, 1); scat_start(1, 1)
    idx_wait(e + 3, 3); scat_wait(1, 1); gather_start(3, 1)
    gather_wait(2, 0); scat_start(2, 0)
    idx_wait(e + 4, 0); scat_wait(2, 0); gather_start(0, 0)
    gather_wait(3, 1); scat_start(3, 1)
    gather_wait(0, 0); scat_start(0, 0)
    scat_wait(1, 1)
    scat_wait(0, 0)

    plsc.subcore_barrier()

    @pl.when(sid < 15)
    def _():
        pltpu.sync_copy(acc.at[pl.ds(sid * _STRIPE, _STRIPE)],
                        out_hbm.at[pl.ds(cid * _N + sid * _STRIPE, _STRIPE)])

    @pl.when(sid == 15)
    def _():
        pltpu.sync_copy(acc.at[pl.ds(_LAST_OFF, _LAST_STRIPE)],
                        out_hbm.at[pl.ds(cid * _N + _LAST_OFF, _LAST_STRIPE)])


def _layer_body(p_ref, x_ref, wrel_ref, wroot_ref, brel_ref,
                batch_ref, h_ref, pool_h_ref, pool_x_ref):
    i = pl.program_id(0)
    h = jnp.dot(p_ref[...], wrel_ref[...], preferred_element_type=jnp.float32)
    h = h + jnp.dot(x_ref[...], wroot_ref[...],
                    preferred_element_type=jnp.float32)
    h = jnp.maximum(h + brel_ref[...], 0.0)
    h_ref[...] = h

    seg = batch_ref[0, 0, :]
    onehot_t = (lax.broadcasted_iota(jnp.int32, (_NSEG, _BN), 0)
                == seg[None, :]).astype(jnp.float32)

    @pl.when(i % (_NB // 2) == 0)
    def _():
        pool_h_ref[...] = jnp.zeros_like(pool_h_ref)
        pool_x_ref[...] = jnp.zeros_like(pool_x_ref)

    pool_h_ref[0] += jnp.dot(onehot_t, h, preferred_element_type=jnp.float32)
    pool_x_ref[0] += jnp.dot(onehot_t, x_ref[...],
                             preferred_element_type=jnp.float32)


def _tc_layer(p, x, wrel, wroot, brel, batch3):
    return pl.pallas_call(
        _layer_body,
        grid=(_NB,),
        in_specs=[
            pl.BlockSpec((_BN, _H), lambda i: (i, 0)),
            pl.BlockSpec((_BN, _H), lambda i: (i, 0)),
            pl.BlockSpec((_H, _H), lambda i: (0, 0)),
            pl.BlockSpec((_H, _H), lambda i: (0, 0)),
            pl.BlockSpec((1, _H), lambda i: (0, 0)),
            pl.BlockSpec((1, 1, _BN), lambda i: (i, 0, 0)),
        ],
        out_specs=[
            pl.BlockSpec((_BN, _H), lambda i: (i, 0)),
            pl.BlockSpec((1, _NSEG, _H), lambda i: (i // (_NB // 2), 0, 0)),
            pl.BlockSpec((1, _NSEG, _H), lambda i: (i // (_NB // 2), 0, 0)),
        ],
        out_shape=[
            jax.ShapeDtypeStruct((2 * _N, _H), jnp.float32),
            jax.ShapeDtypeStruct((2, _NSEG, _H), jnp.float32),
            jax.ShapeDtypeStruct((2, _NSEG, _H), jnp.float32),
        ],
    )(p, x, wrel, wroot, brel, batch3)


def _final_body(ps_ref, pt_ref, w_ref, b_ref, out_ref):
    def embed(p_ref):
        e = jnp.dot(p_ref[...], w_ref[...],
                    preferred_element_type=jnp.float32) + b_ref[...]
        n = jnp.sqrt(jnp.sum(e * e, axis=1, keepdims=True))
        return e / jnp.maximum(n, 1e-12)

    d = embed(ps_ref) - embed(pt_ref)
    out_ref[...] = jnp.sqrt(jnp.sum(d * d, axis=1, keepdims=True))


def _tc_final(ps, pt, w_emb, b_emb2):
    return pl.pallas_call(
        _final_body,
        out_shape=jax.ShapeDtypeStruct((_NSEG, 1), jnp.float32),
    )(ps, pt, w_emb, b_emb2)


def kernel(x_s, edge_index_s, edge_attr_s, x_t, edge_index_t, edge_attr_t,
           x_s_batch, x_t_batch, W_rel0, b_rel0, W_root0, W_rel1, b_rel1,
           W_root1, W_rel2, b_rel2, W_root2, W_emb, b_emb, virtual_embedding):
    del edge_attr_s, edge_attr_t, virtual_embedding
    layers = [(W_rel0, b_rel0.reshape(1, _H), W_root0),
              (W_rel1, b_rel1.reshape(1, _H), W_root1),
              (W_rel2, b_rel2.reshape(1, _H), W_root2)]
    zeros = jnp.zeros((_N, _H), jnp.float32)

    pad = ((0, 0), (0, _EPAD - _EPW))
    src = jnp.concatenate([
        jnp.pad(edge_index_s[0].reshape(16, _EPW), pad),
        jnp.pad((edge_index_t[0] + _N).reshape(16, _EPW), pad),
    ]).reshape(32 * _NCHUNK, _CHUNK)
    dst = jnp.concatenate([
        jnp.pad(edge_index_s[1].reshape(16, _EPW), pad, constant_values=_N),
        jnp.pad(edge_index_t[1].reshape(16, _EPW), pad, constant_values=_N),
    ]).reshape(32 * _NCHUNK, _CHUNK)
    ei = jnp.stack([src, dst], axis=1)
    batch3 = jnp.concatenate([x_s_batch, x_t_batch]).reshape(_NB, 1, _BN)

    x = jnp.concatenate([x_s, x_t], axis=0)
    pooled = []
    h = x
    for l, (wrel, brel, wroot) in enumerate(layers):
        p = _sc_segsum(h, ei, zeros)
        h, pool_h, pool_in = _tc_layer(p, h, wrel, wroot, brel, batch3)
        if l == 0:
            pooled.append(pool_in)
        pooled.append(pool_h)

    ps = jnp.concatenate([q[0] for q in pooled], axis=1)
    pt = jnp.concatenate([q[1] for q in pooled], axis=1)
    geds = _tc_final(ps, pt, W_emb, b_emb.reshape(1, _NSEG))
    return geds.reshape(_NSEG)


# speedup vs baseline: 1.0012x; 1.0012x over previous
import functools

import jax
import jax.numpy as jnp
from jax import lax
from jax.experimental import pallas as pl
from jax.experimental.pallas import tpu as pltpu
from jax.experimental.pallas import tpu_sc as plsc

_N = 10000
_E = 320000
_H = 128
_NSEG = 64
_CHUNK = 96
_EPW = _E // 16
_NCHUNK = -(-_EPW // _CHUNK)
_EPAD = _NCHUNK * _CHUNK
_GPAD = 512

_STRIPE = 632
_LAST_OFF = 15 * _STRIPE
_LAST_STRIPE = _N - _LAST_OFF

_BN = 1000
_NB = (2 * _N) // _BN


@functools.partial(
    pl.kernel,
    mesh=plsc.VectorSubcoreMesh(core_axis_name="c", subcore_axis_name="s"),
    out_type=jax.ShapeDtypeStruct((2 * _N, _H), jnp.float32),
    scratch_types=[
        pltpu.VMEM((2, _CHUNK), jnp.int32),
        pltpu.VMEM((2, _CHUNK), jnp.int32),
        pltpu.VMEM((2, _CHUNK), jnp.int32),
        pltpu.VMEM((2, _CHUNK), jnp.int32),
        pltpu.VMEM((_CHUNK, _H), jnp.float32),
        pltpu.VMEM((_CHUNK, _H), jnp.float32),
        pltpu.VMEM_SHARED((_N + _GPAD, _H), jnp.float32),
        pltpu.SemaphoreType.DMA,
        pltpu.SemaphoreType.DMA,
        pltpu.SemaphoreType.DMA,
        pltpu.SemaphoreType.DMA,
        pltpu.SemaphoreType.DMA,
        pltpu.SemaphoreType.DMA,
        pltpu.SemaphoreType.DMA,
        pltpu.SemaphoreType.DMA,
    ],
)
def _sc_segsum(y_hbm, ei_hbm, zeros_hbm, out_hbm,
               ix0, ix1, ix2, ix3, rows0, rows1, acc,
               si0, si1, si2, si3, sg0, sg1, ss0, ss1):
    cid = lax.axis_index("c")
    sid = lax.axis_index("s")
    wid = cid * 16 + sid
    base = wid * _NCHUNK
    ix = [ix0, ix1, ix2, ix3]
    si = [si0, si1, si2, si3]
    rb = [rows0, rows1]
    sg = [sg0, sg1]
    ss = [ss0, ss1]

    @pl.when(sid < 15)
    def _():
        pltpu.sync_copy(zeros_hbm.at[pl.ds(sid * _STRIPE, _STRIPE)],
                        acc.at[pl.ds(sid * _STRIPE, _STRIPE)])

    @pl.when(sid == 15)
    def _():
        pltpu.sync_copy(zeros_hbm.at[pl.ds(_LAST_OFF, _LAST_STRIPE)],
                        acc.at[pl.ds(_LAST_OFF, _LAST_STRIPE)])

    plsc.subcore_barrier()

    def idx_start(c, k):
        pltpu.make_async_copy(ei_hbm.at[base + c], ix[k], si[k]).start()

    def idx_wait(c, k):
        pltpu.make_async_copy(ei_hbm.at[base + c], ix[k], si[k]).wait()

    def gather_start(k, r):
        pltpu.make_async_copy(y_hbm.at[ix[k].at[0]], rb[r], sg[r]).start()

    def gather_wait(k, r):
        pltpu.make_async_copy(y_hbm.at[ix[k].at[0]], rb[r], sg[r]).wait()

    def scat_start(k, r):
        pltpu.make_async_copy(rb[r], acc.at[ix[k].at[1]],
                              ss[r]).start(add=True)

    def scat_wait(k, r):
        pltpu.make_async_copy(rb[r], acc.at[ix[k].at[1]], ss[r]).wait()

    idx_start(0, 0)
    idx_start(1, 1)
    idx_wait(0, 0); idx_start(2, 2); gather_start(0, 0)
    idx_wait(1, 1); idx_start(3, 3); gather_start(1, 1)
    gather_wait(0, 0); scat_start(0, 0)
    idx_wait(2, 2); scat_wait(0, 0); idx_start(4, 0); gather_start(2, 0)
    gather_wait(1, 1); scat_start(1, 1)
    idx_wait(3, 3); scat_wait(1, 1); idx_start(5, 1); gather_start(3, 1)
    gather_wait(2, 0); scat_start(2, 0)

    def body(m, carry):
        c = 4 * m
        for ki in range(4):
            kr = ki & 1
            idx_wait(c + ki, ki)
            scat_wait(ki - 2 & 3, kr)
            idx_start(c + ki + 2, ki + 2 & 3)
            gather_start(ki, kr)
            gather_wait(ki - 1 & 3, kr ^ 1)
            scat_start(ki - 1 & 3, kr ^ 1)
        return carry

    lax.fori_loop(1, (_NCHUNK - 9) // 4 + 1, body, 0)
    e = _NCHUNK - 5
    idx_wait(e, 0); scat_wait(2, 0); idx_start(e + 2, 2); gather_start(0, 0)
    gather_wait(3, 1); scat_start(3, 1)
    idx_wait(e + 1, 1); scat_wait(3, 1); idx_start(e + 3, 3); gather_start(1, 1)
    gather_wait(0, 0); scat_start(0, 0)
    idx_wait(e + 2, 2); scat_wait(0, 0); idx_start(e + 4, 0); gather_start(2, 0)
    gather_wait(1, 1); scat_start(1, 1)
    idx_wait(e + 3, 3); scat_wait(1, 1); gather_start(3, 1)
    gather_wait(2, 0); scat_start(2, 0)
    idx_wait(e + 4, 0); scat_wait(2, 0); gather_start(0, 0)
    gather_wait(3, 1); scat_start(3, 1)
    gather_wait(0, 0); scat_start(0, 0)
    scat_wait(1, 1)
    scat_wait(0, 0)

    plsc.subcore_barrier()

    @pl.when(sid < 15)
    def _():
        pltpu.sync_copy(acc.at[pl.ds(sid * _STRIPE, _STRIPE)],
                        out_hbm.at[pl.ds(cid * _N + sid * _STRIPE, _STRIPE)])

    @pl.when(sid == 15)
    def _():
        pltpu.sync_copy(acc.at[pl.ds(_LAST_OFF, _LAST_STRIPE)],
                        out_hbm.at[pl.ds(cid * _N + _LAST_OFF, _LAST_STRIPE)])


def _layer_body(p_ref, x_ref, wrel_ref, wroot_ref, brel_ref,
                batch_ref, h_ref, pool_h_ref, pool_x_ref):
    i = pl.program_id(0)
    h = jnp.dot(p_ref[...], wrel_ref[...], preferred_element_type=jnp.float32)
    h = h + jnp.dot(x_ref[...], wroot_ref[...],
                    preferred_element_type=jnp.float32)
    h = jnp.maximum(h + brel_ref[...], 0.0)
    h_ref[...] = h

    seg = batch_ref[0, 0, :]
    onehot_t = (lax.broadcasted_iota(jnp.int32, (_NSEG, _BN), 0)
                == seg[None, :]).astype(jnp.float32)

    @pl.when(i % (_NB // 2) == 0)
    def _():
        pool_h_ref[...] = jnp.zeros_like(pool_h_ref)
        pool_x_ref[...] = jnp.zeros_like(pool_x_ref)

    pool_h_ref[0] += jnp.dot(onehot_t, h, preferred_element_type=jnp.float32)
    pool_x_ref[0] += jnp.dot(onehot_t, x_ref[...],
                             preferred_element_type=jnp.float32)


def _tc_layer(p, x, wrel, wroot, brel, batch3):
    return pl.pallas_call(
        _layer_body,
        grid=(_NB,),
        in_specs=[
            pl.BlockSpec((_BN, _H), lambda i: (i, 0)),
            pl.BlockSpec((_BN, _H), lambda i: (i, 0)),
            pl.BlockSpec((_H, _H), lambda i: (0, 0)),
            pl.BlockSpec((_H, _H), lambda i: (0, 0)),
            pl.BlockSpec((1, _H), lambda i: (0, 0)),
            pl.BlockSpec((1, 1, _BN), lambda i: (i, 0, 0)),
        ],
        out_specs=[
            pl.BlockSpec((_BN, _H), lambda i: (i, 0)),
            pl.BlockSpec((1, _NSEG, _H), lambda i: (i // (_NB // 2), 0, 0)),
            pl.BlockSpec((1, _NSEG, _H), lambda i: (i // (_NB // 2), 0, 0)),
        ],
        out_shape=[
            jax.ShapeDtypeStruct((2 * _N, _H), jnp.float32),
            jax.ShapeDtypeStruct((2, _NSEG, _H), jnp.float32),
            jax.ShapeDtypeStruct((2, _NSEG, _H), jnp.float32),
        ],
    )(p, x, wrel, wroot, brel, batch3)


def _final_body(ps_ref, pt_ref, w_ref, b_ref, out_ref):
    def embed(p_ref):
        e = jnp.dot(p_ref[...], w_ref[...],
                    preferred_element_type=jnp.float32) + b_ref[...]
        n = jnp.sqrt(jnp.sum(e * e, axis=1, keepdims=True))
        return e / jnp.maximum(n, 1e-12)

    d = embed(ps_ref) - embed(pt_ref)
    out_ref[...] = jnp.sqrt(jnp.sum(d * d, axis=1, keepdims=True))


def _tc_final(ps, pt, w_emb, b_emb2):
    return pl.pallas_call(
        _final_body,
        out_shape=jax.ShapeDtypeStruct((_NSEG, 1), jnp.float32),
    )(ps, pt, w_emb, b_emb2)


def kernel(x_s, edge_index_s, edge_attr_s, x_t, edge_index_t, edge_attr_t,
           x_s_batch, x_t_batch, W_rel0, b_rel0, W_root0, W_rel1, b_rel1,
           W_root1, W_rel2, b_rel2, W_root2, W_emb, b_emb, virtual_embedding):
    del edge_attr_s, edge_attr_t, virtual_embedding
    layers = [(W_rel0, b_rel0.reshape(1, _H), W_root0),
              (W_rel1, b_rel1.reshape(1, _H), W_root1),
              (W_rel2, b_rel2.reshape(1, _H), W_root2)]
    zeros = jnp.zeros((_N, _H), jnp.float32)

    pad = ((0, 0), (0, _EPAD - _EPW))
    dpad = jnp.broadcast_to(
        _N + (jnp.arange(_EPAD - _EPW, dtype=jnp.int32) % _GPAD),
        (16, _EPAD - _EPW))
    src = jnp.concatenate([
        jnp.pad(edge_index_s[0].reshape(16, _EPW), pad),
        jnp.pad((edge_index_t[0] + _N).reshape(16, _EPW), pad),
    ]).reshape(32 * _NCHUNK, _CHUNK)
    dst = jnp.concatenate([
        jnp.concatenate([edge_index_s[1].reshape(16, _EPW), dpad], axis=1),
        jnp.concatenate([edge_index_t[1].reshape(16, _EPW), dpad], axis=1),
    ]).reshape(32 * _NCHUNK, _CHUNK)
    ei = jnp.stack([src, dst], axis=1)
    batch3 = jnp.concatenate([x_s_batch, x_t_batch]).reshape(_NB, 1, _BN)

    x = jnp.concatenate([x_s, x_t], axis=0)
    pooled = []
    h = x
    for l, (wrel, brel, wroot) in enumerate(layers):
        p = _sc_segsum(h, ei, zeros)
        h, pool_h, pool_in = _tc_layer(p, h, wrel, wroot, brel, batch3)
        if l == 0:
            pooled.append(pool_in)
        pooled.append(pool_h)

    ps = jnp.concatenate([q[0] for q in pooled], axis=1)
    pt = jnp.concatenate([q[1] for q in pooled], axis=1)
    geds = _tc_final(ps, pt, W_emb, b_emb.reshape(1, _NSEG))
    return geds.reshape(_NSEG)


# speedup vs baseline: 1.2073x; 1.2058x over previous
import functools

import jax
import jax.numpy as jnp
from jax import lax
from jax.experimental import pallas as pl
from jax.experimental.pallas import tpu as pltpu
from jax.experimental.pallas import tpu_sc as plsc

_N = 10000
_E = 320000
_H = 128
_NSEG = 64
_CHUNK = 80
_EPW = _E // 16
_NCHUNK = -(-_EPW // _CHUNK)
_EPAD = _NCHUNK * _CHUNK
_GPAD = 512

_STRIPE = 632
_LAST_OFF = 15 * _STRIPE
_LAST_STRIPE = _N - _LAST_OFF

_BN = 1000
_NB = (2 * _N) // _BN


@functools.partial(
    pl.kernel,
    mesh=plsc.VectorSubcoreMesh(core_axis_name="c", subcore_axis_name="s"),
    out_type=jax.ShapeDtypeStruct((2 * _N, _H), jnp.float32),
    scratch_types=[
        pltpu.VMEM((2, _CHUNK), jnp.int32),
        pltpu.VMEM((2, _CHUNK), jnp.int32),
        pltpu.VMEM((2, _CHUNK), jnp.int32),
        pltpu.VMEM((2, _CHUNK), jnp.int32),
        pltpu.VMEM((_CHUNK, _H), jnp.float32),
        pltpu.VMEM((_CHUNK, _H), jnp.float32),
        pltpu.VMEM_SHARED((_N + _GPAD, _H), jnp.float32),
        pltpu.SemaphoreType.DMA,
        pltpu.SemaphoreType.DMA,
        pltpu.SemaphoreType.DMA,
        pltpu.SemaphoreType.DMA,
        pltpu.SemaphoreType.DMA,
        pltpu.SemaphoreType.DMA,
        pltpu.SemaphoreType.DMA,
        pltpu.SemaphoreType.DMA,
    ],
)
def _sc_segsum(y_hbm, ei_hbm, zeros_hbm, out_hbm,
               ix0, ix1, ix2, ix3, rows0, rows1, acc,
               si0, si1, si2, si3, sg0, sg1, ss0, ss1):
    cid = lax.axis_index("c")
    sid = lax.axis_index("s")
    wid = cid * 16 + sid
    base = wid * _NCHUNK
    ix = [ix0, ix1, ix2, ix3]
    si = [si0, si1, si2, si3]
    rb = [rows0, rows1]
    sg = [sg0, sg1]
    ss = [ss0, ss1]

    @pl.when(sid < 15)
    def _():
        pltpu.sync_copy(zeros_hbm.at[pl.ds(sid * _STRIPE, _STRIPE)],
                        acc.at[pl.ds(sid * _STRIPE, _STRIPE)])

    @pl.when(sid == 15)
    def _():
        pltpu.sync_copy(zeros_hbm.at[pl.ds(_LAST_OFF, _LAST_STRIPE)],
                        acc.at[pl.ds(_LAST_OFF, _LAST_STRIPE)])

    plsc.subcore_barrier()

    def idx_start(c, k):
        pltpu.make_async_copy(ei_hbm.at[base + c], ix[k], si[k]).start()

    def idx_wait(c, k):
        pltpu.make_async_copy(ei_hbm.at[base + c], ix[k], si[k]).wait()

    def gather_start(k, r):
        pltpu.make_async_copy(y_hbm.at[ix[k].at[0]], rb[r], sg[r]).start()

    def gather_wait(k, r):
        pltpu.make_async_copy(y_hbm.at[ix[k].at[0]], rb[r], sg[r]).wait()

    def scat_start(k, r):
        pltpu.make_async_copy(rb[r], acc.at[ix[k].at[1]],
                              ss[r]).start(add=True)

    def scat_wait(k, r):
        pltpu.make_async_copy(rb[r], acc.at[ix[k].at[1]], ss[r]).wait()

    idx_start(0, 0)
    idx_start(1, 1)
    idx_wait(0, 0); idx_start(2, 2); gather_start(0, 0)
    idx_wait(1, 1); idx_start(3, 3); gather_start(1, 1)
    gather_wait(0, 0); scat_start(0, 0)
    idx_wait(2, 2); scat_wait(0, 0); idx_start(4, 0); gather_start(2, 0)
    gather_wait(1, 1); scat_start(1, 1)
    idx_wait(3, 3); scat_wait(1, 1); idx_start(5, 1); gather_start(3, 1)
    gather_wait(2, 0); scat_start(2, 0)

    def body(m, carry):
        c = 4 * m
        for ki in range(4):
            kr = ki & 1
            idx_wait(c + ki, ki)
            scat_wait(ki - 2 & 3, kr)
            idx_start(c + ki + 2, ki + 2 & 3)
            gather_start(ki, kr)
            gather_wait(ki - 1 & 3, kr ^ 1)
            scat_start(ki - 1 & 3, kr ^ 1)
        return carry

    _R = 4 + _NCHUNK % 4
    lax.fori_loop(1, (_NCHUNK - _R) // 4, body, 0)
    for c in range(_NCHUNK - _R, _NCHUNK):
        ki, kr = c % 4, c % 2
        idx_wait(c, ki)
        scat_wait(ki - 2 & 3, kr)
        if c + 2 < _NCHUNK:
            idx_start(c + 2, ki + 2 & 3)
        gather_start(ki, kr)
        gather_wait(ki - 1 & 3, kr ^ 1)
        scat_start(ki - 1 & 3, kr ^ 1)
    _L = _NCHUNK - 1
    gather_wait(_L % 4, _L % 2)
    scat_start(_L % 4, _L % 2)
    scat_wait(_L - 1 & 3, _L - 1 & 1)
    scat_wait(_L % 4, _L % 2)

    plsc.subcore_barrier()

    @pl.when(sid < 15)
    def _():
        pltpu.sync_copy(acc.at[pl.ds(sid * _STRIPE, _STRIPE)],
                        out_hbm.at[pl.ds(cid * _N + sid * _STRIPE, _STRIPE)])

    @pl.when(sid == 15)
    def _():
        pltpu.sync_copy(acc.at[pl.ds(_LAST_OFF, _LAST_STRIPE)],
                        out_hbm.at[pl.ds(cid * _N + _LAST_OFF, _LAST_STRIPE)])


def _layer_body(p_ref, x_ref, wrel_ref, wroot_ref, brel_ref,
                batch_ref, h_ref, pool_h_ref, pool_x_ref):
    i = pl.program_id(0)
    h = jnp.dot(p_ref[...], wrel_ref[...], preferred_element_type=jnp.float32)
    h = h + jnp.dot(x_ref[...], wroot_ref[...],
                    preferred_element_type=jnp.float32)
    h = jnp.maximum(h + brel_ref[...], 0.0)
    h_ref[...] = h

    seg = batch_ref[0, 0, :]
    onehot_t = (lax.broadcasted_iota(jnp.int32, (_NSEG, _BN), 0)
                == seg[None, :]).astype(jnp.float32)

    @pl.when(i % (_NB // 2) == 0)
    def _():
        pool_h_ref[...] = jnp.zeros_like(pool_h_ref)
        pool_x_ref[...] = jnp.zeros_like(pool_x_ref)

    pool_h_ref[0] += jnp.dot(onehot_t, h, preferred_element_type=jnp.float32)
    pool_x_ref[0] += jnp.dot(onehot_t, x_ref[...],
                             preferred_element_type=jnp.float32)


def _tc_layer(p, x, wrel, wroot, brel, batch3):
    return pl.pallas_call(
        _layer_body,
        grid=(_NB,),
        in_specs=[
            pl.BlockSpec((_BN, _H), lambda i: (i, 0)),
            pl.BlockSpec((_BN, _H), lambda i: (i, 0)),
            pl.BlockSpec((_H, _H), lambda i: (0, 0)),
            pl.BlockSpec((_H, _H), lambda i: (0, 0)),
            pl.BlockSpec((1, _H), lambda i: (0, 0)),
            pl.BlockSpec((1, 1, _BN), lambda i: (i, 0, 0)),
        ],
        out_specs=[
            pl.BlockSpec((_BN, _H), lambda i: (i, 0)),
            pl.BlockSpec((1, _NSEG, _H), lambda i: (i // (_NB // 2), 0, 0)),
            pl.BlockSpec((1, _NSEG, _H), lambda i: (i // (_NB // 2), 0, 0)),
        ],
        out_shape=[
            jax.ShapeDtypeStruct((2 * _N, _H), jnp.float32),
            jax.ShapeDtypeStruct((2, _NSEG, _H), jnp.float32),
            jax.ShapeDtypeStruct((2, _NSEG, _H), jnp.float32),
        ],
    )(p, x, wrel, wroot, brel, batch3)


def _final_body(ps_ref, pt_ref, w_ref, b_ref, out_ref):
    def embed(p_ref):
        e = jnp.dot(p_ref[...], w_ref[...],
                    preferred_element_type=jnp.float32) + b_ref[...]
        n = jnp.sqrt(jnp.sum(e * e, axis=1, keepdims=True))
        return e / jnp.maximum(n, 1e-12)

    d = embed(ps_ref) - embed(pt_ref)
    out_ref[...] = jnp.sqrt(jnp.sum(d * d, axis=1, keepdims=True))


def _tc_final(ps, pt, w_emb, b_emb2):
    return pl.pallas_call(
        _final_body,
        out_shape=jax.ShapeDtypeStruct((_NSEG, 1), jnp.float32),
    )(ps, pt, w_emb, b_emb2)


def kernel(x_s, edge_index_s, edge_attr_s, x_t, edge_index_t, edge_attr_t,
           x_s_batch, x_t_batch, W_rel0, b_rel0, W_root0, W_rel1, b_rel1,
           W_root1, W_rel2, b_rel2, W_root2, W_emb, b_emb, virtual_embedding):
    del edge_attr_s, edge_attr_t, virtual_embedding
    layers = [(W_rel0, b_rel0.reshape(1, _H), W_root0),
              (W_rel1, b_rel1.reshape(1, _H), W_root1),
              (W_rel2, b_rel2.reshape(1, _H), W_root2)]
    zeros = jnp.zeros((_N, _H), jnp.float32)

    pad = ((0, 0), (0, _EPAD - _EPW))
    dpad = jnp.broadcast_to(
        _N + (jnp.arange(_EPAD - _EPW, dtype=jnp.int32) % _GPAD),
        (16, _EPAD - _EPW))
    src = jnp.concatenate([
        jnp.pad(edge_index_s[0].reshape(16, _EPW), pad),
        jnp.pad((edge_index_t[0] + _N).reshape(16, _EPW), pad),
    ]).reshape(32 * _NCHUNK, _CHUNK)
    dst = jnp.concatenate([
        jnp.concatenate([edge_index_s[1].reshape(16, _EPW), dpad], axis=1),
        jnp.concatenate([edge_index_t[1].reshape(16, _EPW), dpad], axis=1),
    ]).reshape(32 * _NCHUNK, _CHUNK)
    ei = jnp.stack([src, dst], axis=1)
    batch3 = jnp.concatenate([x_s_batch, x_t_batch]).reshape(_NB, 1, _BN)

    x = jnp.concatenate([x_s, x_t], axis=0)
    pooled = []
    h = x
    for l, (wrel, brel, wroot) in enumerate(layers):
        p = _sc_segsum(h, ei, zeros)
        h, pool_h, pool_in = _tc_layer(p, h, wrel, wroot, brel, batch3)
        if l == 0:
            pooled.append(pool_in)
        pooled.append(pool_h)

    ps = jnp.concatenate([q[0] for q in pooled], axis=1)
    pt = jnp.concatenate([q[1] for q in pooled], axis=1)
    geds = _tc_final(ps, pt, W_emb, b_emb.reshape(1, _NSEG))
    return geds.reshape(_NSEG)
